# baseline (speedup 1.0000x reference)
import functools
import jax
import jax.numpy as jnp
from jax import lax
from jax.experimental import pallas as pl
from jax.experimental.pallas import tpu as pltpu
from jax.experimental.pallas import tpu_sc as plsc

N = 50000
F_IN = 16
E = 40000
S = 10000
EO = 20000
G = 256

NPAD = 51200
SPAD = 10240
EP1 = 40960
EP2 = 20480

@functools.cache
def _sc_mesh():
    return plsc.VectorSubcoreMesh(
        core_axis_name="c", subcore_axis_name="s",
        num_cores=2, num_subcores=16)


@functools.partial(jax.jit, static_argnames=("rows_per_worker",))
def _sc_gather(tab, idx, rows_per_worker):
    ep, d = idx.shape[0], tab.shape[1]
    nchunks = rows_per_worker // 128

    def body(tab_ref, idx_ref, out_ref, idx_v, rows_v, sem):
        wid = lax.axis_index("s") * 2 + lax.axis_index("c")
        base = wid * rows_per_worker

        def step(ci, carry):
            off = base + ci * 128
            pltpu.sync_copy(idx_ref.at[pl.ds(off, 128)], idx_v)
            pltpu.async_copy(tab_ref.at[idx_v], rows_v, sem).wait()
            pltpu.sync_copy(rows_v, out_ref.at[pl.ds(off, 128)])
            return carry

        lax.fori_loop(0, nchunks, step, 0, unroll=False)

    return pl.kernel(
        body,
        out_type=jax.ShapeDtypeStruct((ep, d), jnp.float32),
        mesh=_sc_mesh(),
        compiler_params=pltpu.CompilerParams(use_tc_tiling_on_sc=False),
        scratch_types=[
            pltpu.VMEM((128,), jnp.int32),
            pltpu.VMEM((128, d), jnp.float32),
            pltpu.SemaphoreType.DMA,
        ],
    )(tab, idx)


@functools.partial(jax.jit, static_argnames=("tpad", "with_counts"))
def _sc_scatter_add(vals, idx, tpad, with_counts):
    rp, d = vals.shape
    t2 = tpad // 2
    acc_rows = ((t2 + 1 + 127) // 128) * 128
    nzc = acc_rows // 128
    zc_per_tile = (nzc + 15) // 16
    rows_per_tile = rp // 16
    in_chunks = rows_per_tile // 128
    out_rows_per_tile = t2 // 16

    def body(vals_ref, idx_ref, zero_ref, z16_ref, one_ref, *refs):
        if with_counts:
            (out_ref, cnt_ref, idx_v, rows_v, zero_v, z16_v, ones_v,
             acc, accc) = refs
        else:
            out_ref, idx_v, rows_v, zero_v, acc = refs
        c = lax.axis_index("c")
        sid = lax.axis_index("s")
        base_t = c * t2

        pltpu.sync_copy(zero_ref, zero_v)
        if with_counts:
            pltpu.sync_copy(one_ref, ones_v)
            pltpu.sync_copy(z16_ref, z16_v)

        def zstep(j, carry):
            ch = sid * zc_per_tile + j

            @pl.when(ch < nzc)
            def _():
                pltpu.sync_copy(zero_v, acc.at[pl.ds(ch * 128, 128)])
                if with_counts:
                    pltpu.sync_copy(z16_v, accc.at[pl.ds(ch * 128, 128)])
            return carry

        lax.fori_loop(0, zc_per_tile, zstep, 0, unroll=False)
        plsc.subcore_barrier()

        def step(ci, carry):
            off = sid * rows_per_tile + ci * 128
            pltpu.sync_copy(idx_ref.at[pl.ds(off, 128)], idx_v)
            pltpu.sync_copy(vals_ref.at[pl.ds(off, 128)], rows_v)
            for j in range(8):
                v = idx_v[pl.ds(j * 16, 16)]
                loc = v - base_t
                bad = (loc < 0) | (loc >= t2)
                idx_v[pl.ds(j * 16, 16)] = jnp.where(bad, t2, loc)
            pltpu.sync_copy(rows_v, acc.at[idx_v], add=True)
            if with_counts:
                pltpu.sync_copy(ones_v, accc.at[idx_v], add=True)
            return carry

        lax.fori_loop(0, in_chunks, step, 0, unroll=False)
        plsc.subcore_barrier()

        ob = sid * out_rows_per_tile
        pltpu.sync_copy(acc.at[pl.ds(ob, out_rows_per_tile)],
                        out_ref.at[pl.ds(base_t + ob, out_rows_per_tile)])
        if with_counts:
            pltpu.sync_copy(accc.at[pl.ds(ob, out_rows_per_tile)],
                            cnt_ref.at[pl.ds(base_t + ob, out_rows_per_tile)])

    out_type = [jax.ShapeDtypeStruct((tpad, d), jnp.float32)]
    scratch = [
        pltpu.VMEM((128,), jnp.int32),
        pltpu.VMEM((128, d), jnp.float32),
        pltpu.VMEM((128, d), jnp.float32),
    ]
    if with_counts:
        out_type.append(jax.ShapeDtypeStruct((tpad, 16), jnp.float32))
        scratch.append(pltpu.VMEM((128, 16), jnp.float32))
        scratch.append(pltpu.VMEM((128, 16), jnp.float32))
        scratch.append(pltpu.VMEM_SHARED((acc_rows, d), jnp.float32))
        scratch.append(pltpu.VMEM_SHARED((acc_rows, 16), jnp.float32))
    else:
        scratch.append(pltpu.VMEM_SHARED((acc_rows, d), jnp.float32))

    zeros = jnp.zeros((128, d), jnp.float32)
    zeros16 = jnp.zeros((128, 16), jnp.float32)
    ones = jnp.ones((128, 16), jnp.float32)
    res = pl.kernel(
        body,
        out_type=tuple(out_type),
        mesh=_sc_mesh(),
        compiler_params=pltpu.CompilerParams(use_tc_tiling_on_sc=False),
        scratch_types=scratch,
    )(vals, idx, zeros, zeros16, ones)
    return res if with_counts else res[0]


def _make_msg_body(i_dim, o_dim):
    def body(xj_ref, ea_ref, w1_ref, b1_ref, w2_ref, b2_ref, out_ref):
        xj = xj_ref[...]
        h = jnp.maximum(
            jnp.dot(ea_ref[...], w1_ref[...],
                    preferred_element_type=jnp.float32) + b1_ref[...], 0.0)
        W = jnp.dot(h, w2_ref[...],
                    preferred_element_type=jnp.float32) + b2_ref[...]
        blk = xj.shape[0]
        W3 = W.reshape(blk, i_dim, 128)
        s = jnp.sum(W3 * xj[:, :, None], axis=1)
        out_ref[...] = s[:, :o_dim]
    return body


@functools.partial(jax.jit, static_argnames=("blk", "o"))
def _tc_msg(xj, ea8, w18, b1, w2p, b2p, blk, o):
    ep, i = xj.shape
    grid = ep // blk
    return pl.pallas_call(
        _make_msg_body(i, o),
        grid=(grid,),
        in_specs=[
            pl.BlockSpec((blk, i), lambda b: (b, 0)),
            pl.BlockSpec((blk, 8), lambda b: (b, 0)),
            pl.BlockSpec((8, 128), lambda b: (0, 0)),
            pl.BlockSpec((1, 128), lambda b: (0, 0)),
            pl.BlockSpec((128, i * 128), lambda b: (0, 0)),
            pl.BlockSpec((1, i * 128), lambda b: (0, 0)),
        ],
        out_specs=pl.BlockSpec((blk, o), lambda b: (b, 0)),
        out_shape=jax.ShapeDtypeStruct((ep, o), jnp.float32),
    )(xj, ea8, w18, b1, w2p, b2p)


def _elu(z):
    return jnp.where(z > 0.0, z, jnp.exp(jnp.minimum(z, 0.0)) - 1.0)


def _combine_body(agg_ref, x_ref, root_ref, bias_ref, out_ref):
    z = (agg_ref[...]
         + jnp.dot(x_ref[...], root_ref[...],
                   preferred_element_type=jnp.float32) + bias_ref[...])
    out_ref[...] = _elu(z)


@functools.partial(jax.jit, static_argnames=("blk",))
def _tc_combine(agg, x, root, bias2, blk):
    npad, o = agg.shape
    i = x.shape[1]
    return pl.pallas_call(
        _combine_body,
        grid=(npad // blk,),
        in_specs=[
            pl.BlockSpec((blk, o), lambda b: (b, 0)),
            pl.BlockSpec((blk, i), lambda b: (b, 0)),
            pl.BlockSpec((i, o), lambda b: (0, 0)),
            pl.BlockSpec((1, o), lambda b: (0, 0)),
        ],
        out_specs=pl.BlockSpec((blk, o), lambda b: (b, 0)),
        out_shape=jax.ShapeDtypeStruct((npad, o), jnp.float32),
    )(agg, x, root, bias2)


def _div_body(s_ref, c_ref, out_ref):
    cnt = jnp.maximum(c_ref[:, 0:1], 1.0)
    out_ref[...] = s_ref[...] / cnt


@jax.jit
def _tc_divide(sums, counts):
    spad, d = sums.shape
    blk = 1024
    return pl.pallas_call(
        _div_body,
        grid=(spad // blk,),
        in_specs=[
            pl.BlockSpec((blk, d), lambda b: (b, 0)),
            pl.BlockSpec((blk, 16), lambda b: (b, 0)),
        ],
        out_specs=pl.BlockSpec((blk, d), lambda b: (b, 0)),
        out_shape=jax.ShapeDtypeStruct((spad, d), jnp.float32),
    )(sums, counts)


def _head_body(seg_ref, x_ref, f1w_ref, f1b_ref, f2w_ref, f2b_ref,
               f3w_ref, f3b_ref, out_ref, acc_s, acc_c):
    pid = pl.program_id(0)
    nsteps = pl.num_programs(0)

    @pl.when(pid == 0)
    def _():
        acc_s[...] = jnp.zeros_like(acc_s)
        acc_c[...] = jnp.zeros_like(acc_c)

    seg = seg_ref[0, 0, :]
    blk = seg.shape[0]
    seg2 = jnp.broadcast_to(seg[None, :], (G, blk))
    gids = lax.broadcasted_iota(jnp.int32, (G, blk), 0)
    oh = (seg2 == gids).astype(jnp.float32)
    xb = x_ref[...]
    acc_s[...] += jnp.dot(oh, xb, preferred_element_type=jnp.float32)
    acc_c[...] += jnp.dot(oh, jnp.ones_like(xb),
                          preferred_element_type=jnp.float32)

    @pl.when(pid == nsteps - 1)
    def _():
        mean = acc_s[...] / jnp.maximum(acc_c[...], 1.0)
        z1 = _elu(jnp.dot(mean, f1w_ref[...],
                          preferred_element_type=jnp.float32) + f1b_ref[...])
        z2 = _elu(jnp.dot(z1, f2w_ref[...],
                          preferred_element_type=jnp.float32) + f2b_ref[...])
        out_ref[...] = (jnp.dot(z2, f3w_ref[...],
                                preferred_element_type=jnp.float32)
                        + f3b_ref[...])


@jax.jit
def _tc_head(seg3, x2, f1w, f1b, f2w, f2b, f3w8, f3b8):
    blk = 1024
    nblk = seg3.shape[0]
    return pl.pallas_call(
        _head_body,
        grid=(nblk,),
        in_specs=[
            pl.BlockSpec((1, 1, blk), lambda b: (b, 0, 0)),
            pl.BlockSpec((blk, 64), lambda b: (b, 0)),
            pl.BlockSpec((64, 32), lambda b: (0, 0)),
            pl.BlockSpec((1, 32), lambda b: (0, 0)),
            pl.BlockSpec((32, 16), lambda b: (0, 0)),
            pl.BlockSpec((1, 16), lambda b: (0, 0)),
            pl.BlockSpec((16, 8), lambda b: (0, 0)),
            pl.BlockSpec((1, 8), lambda b: (0, 0)),
        ],
        out_specs=pl.BlockSpec((G, 8), lambda b: (0, 0)),
        out_shape=jax.ShapeDtypeStruct((G, 8), jnp.float32),
        scratch_shapes=[
            pltpu.VMEM((G, 64), jnp.float32),
            pltpu.VMEM((G, 64), jnp.float32),
        ],
    )(seg3, x2, f1w, f1b, f2w, f2b, f3w8, f3b8)


def _nnconv_layer(x, src, dst, ea8, w1, b1, w2, b2, root, bias, tpad, blk):
    i, o = root.shape
    w18 = jnp.zeros((8, 128), jnp.float32).at[:5].set(w1)
    b1r = b1.reshape(1, 128)
    w2p = jnp.zeros((128, i, 128), jnp.float32)
    w2p = w2p.at[:, :, :o].set(w2.reshape(128, i, o)).reshape(128, i * 128)
    b2p = jnp.zeros((i, 128), jnp.float32)
    b2p = b2p.at[:, :o].set(b2.reshape(i, o)).reshape(1, i * 128)
    xj = _sc_gather(x, src, src.shape[0] // 32)
    msg = _tc_msg(xj, ea8, w18, b1r, w2p, b2p, blk, o)
    agg = _sc_scatter_add(msg, dst, tpad, False)
    return _tc_combine(agg, x, root, bias.reshape(1, o), 512)


def kernel(x, edge_index, edge_attr, node_to_subgraph, original_edge_index,
           original_edge_attr, subgraph_to_graph,
           sub0_nw1, sub0_nb1, sub0_nw2, sub0_nb2, sub0_root, sub0_bias,
           sub1_nw1, sub1_nb1, sub1_nw2, sub1_nb2, sub1_root, sub1_bias,
           sub2_nw1, sub2_nb1, sub2_nw2, sub2_nb2, sub2_root, sub2_bias,
           gl0_nw1, gl0_nb1, gl0_nw2, gl0_nb2, gl0_root, gl0_bias,
           gl1_nw1, gl1_nb1, gl1_nw2, gl1_nb2, gl1_root, gl1_bias,
           fc1_w, fc1_b, fc2_w, fc2_b, fc3_w, fc3_b):
    f32 = jnp.float32
    x = x.astype(f32)

    src = jnp.pad(edge_index[0].astype(jnp.int32), (0, EP1 - E))
    dst = jnp.pad(edge_index[1].astype(jnp.int32), (0, EP1 - E),
                  constant_values=NPAD)
    ea8 = jnp.zeros((EP1, 8), f32).at[:E, :5].set(edge_attr)
    osrc = jnp.pad(original_edge_index[0].astype(jnp.int32), (0, EP2 - EO))
    odst = jnp.pad(original_edge_index[1].astype(jnp.int32), (0, EP2 - EO),
                   constant_values=SPAD)
    oea8 = jnp.zeros((EP2, 8), f32).at[:EO, :5].set(original_edge_attr)
    n2s = jnp.pad(node_to_subgraph.astype(jnp.int32), (0, NPAD - N),
                  constant_values=SPAD)
    s2g = jnp.pad(subgraph_to_graph.astype(jnp.int32), (0, SPAD - S),
                  constant_values=G)

    xp = jnp.zeros((NPAD, F_IN), f32).at[:N].set(x)

    subs = [
        (sub0_nw1, sub0_nb1, sub0_nw2, sub0_nb2, sub0_root, sub0_bias),
        (sub1_nw1, sub1_nb1, sub1_nw2, sub1_nb2, sub1_root, sub1_bias),
        (sub2_nw1, sub2_nb1, sub2_nw2, sub2_nb2, sub2_root, sub2_bias),
    ]
    h = xp
    for (w1, b1, w2, b2, root, bias) in subs:
        h = _nnconv_layer(h, src, dst, ea8, w1, b1, w2, b2, root, bias,
                          NPAD, 256)

    sums, counts = _sc_scatter_add(h, n2s, SPAD, True)
    p = _tc_divide(sums, counts)

    for (w1, b1, w2, b2, root, bias) in [
            (gl0_nw1, gl0_nb1, gl0_nw2, gl0_nb2, gl0_root, gl0_bias),
            (gl1_nw1, gl1_nb1, gl1_nw2, gl1_nb2, gl1_root, gl1_bias)]:
        p = _nnconv_layer(p, osrc, odst, oea8, w1, b1, w2, b2, root, bias,
                          SPAD, 256)

    seg3 = s2g.reshape(SPAD // 1024, 1, 1024)
    f3w8 = jnp.zeros((16, 8), f32).at[:, 0:1].set(fc3_w)
    f3b8 = jnp.zeros((1, 8), f32).at[0, 0].set(fc3_b[0])
    out8 = _tc_head(seg3, p, fc1_w, fc1_b.reshape(1, 32),
                    fc2_w, fc2_b.reshape(1, 16), f3w8, f3b8)
    return out8[:, 0]


# speedup vs baseline: 1.2513x; 1.2513x over previous
import functools
import jax
import jax.numpy as jnp
from jax import lax
from jax.experimental import pallas as pl
from jax.experimental.pallas import tpu as pltpu
from jax.experimental.pallas import tpu_sc as plsc

N = 50000
F_IN = 16
E = 40000
S = 10000
EO = 20000
G = 256

NPAD = 51200
SPAD = 10240
EP1 = 40960
EP2 = 20480

@functools.cache
def _sc_mesh():
    return plsc.VectorSubcoreMesh(
        core_axis_name="c", subcore_axis_name="s",
        num_cores=2, num_subcores=16)


@functools.partial(jax.jit, static_argnames=("rows_per_worker",))
def _sc_gather(tab, idx, rows_per_worker):
    ep, d = idx.shape[0], tab.shape[1]
    chunk = 128 if rows_per_worker % 128 == 0 else 64
    nchunks = rows_per_worker // chunk

    def body(tab_ref, idx_ref, out_ref, idx_v, rows_v, sem):
        wid = lax.axis_index("s") * 2 + lax.axis_index("c")
        base = wid * rows_per_worker

        def step(ci, carry):
            off = base + ci * chunk
            pltpu.sync_copy(idx_ref.at[pl.ds(off, chunk)], idx_v)
            pltpu.async_copy(tab_ref.at[idx_v], rows_v, sem).wait()
            pltpu.sync_copy(rows_v, out_ref.at[pl.ds(off, chunk)])
            return carry

        lax.fori_loop(0, nchunks, step, 0, unroll=False)

    return pl.kernel(
        body,
        out_type=jax.ShapeDtypeStruct((ep, d), jnp.float32),
        mesh=_sc_mesh(),
        compiler_params=pltpu.CompilerParams(use_tc_tiling_on_sc=False),
        scratch_types=[
            pltpu.VMEM((chunk,), jnp.int32),
            pltpu.VMEM((chunk, d), jnp.float32),
            pltpu.SemaphoreType.DMA,
        ],
    )(tab, idx)


@functools.partial(jax.jit, static_argnames=("tpad", "with_counts"))
def _sc_scatter_add(vals, idx, tpad, with_counts):
    rp, d = vals.shape
    t2 = tpad // 2
    acc_rows = ((t2 + 1 + 127) // 128) * 128
    nzc = acc_rows // 128
    zc_per_tile = (nzc + 15) // 16
    rows_per_tile = rp // 16
    in_chunks = rows_per_tile // 128
    out_rows_per_tile = t2 // 16

    def body(vals_ref, idx_ref, zero_ref, z16_ref, one_ref, *refs):
        if with_counts:
            (out_ref, cnt_ref, idx_v, rows_v, zero_v, z16_v, ones_v,
             acc, accc) = refs
        else:
            out_ref, idx_v, rows_v, zero_v, acc = refs
        c = lax.axis_index("c")
        sid = lax.axis_index("s")
        base_t = c * t2

        pltpu.sync_copy(zero_ref, zero_v)
        if with_counts:
            pltpu.sync_copy(one_ref, ones_v)
            pltpu.sync_copy(z16_ref, z16_v)

        def zstep(j, carry):
            ch = sid * zc_per_tile + j

            @pl.when(ch < nzc)
            def _():
                pltpu.sync_copy(zero_v, acc.at[pl.ds(ch * 128, 128)])
                if with_counts:
                    pltpu.sync_copy(z16_v, accc.at[pl.ds(ch * 128, 128)])
            return carry

        lax.fori_loop(0, zc_per_tile, zstep, 0, unroll=False)
        plsc.subcore_barrier()

        def step(ci, carry):
            off = sid * rows_per_tile + ci * 128
            pltpu.sync_copy(idx_ref.at[pl.ds(off, 128)], idx_v)
            pltpu.sync_copy(vals_ref.at[pl.ds(off, 128)], rows_v)
            for j in range(8):
                v = idx_v[pl.ds(j * 16, 16)]
                loc = v - base_t
                bad = (loc < 0) | (loc >= t2)
                idx_v[pl.ds(j * 16, 16)] = jnp.where(bad, t2, loc)
            pltpu.sync_copy(rows_v, acc.at[idx_v], add=True)
            if with_counts:
                pltpu.sync_copy(ones_v, accc.at[idx_v], add=True)
            return carry

        lax.fori_loop(0, in_chunks, step, 0, unroll=False)
        plsc.subcore_barrier()

        ob = sid * out_rows_per_tile
        pltpu.sync_copy(acc.at[pl.ds(ob, out_rows_per_tile)],
                        out_ref.at[pl.ds(base_t + ob, out_rows_per_tile)])
        if with_counts:
            pltpu.sync_copy(accc.at[pl.ds(ob, out_rows_per_tile)],
                            cnt_ref.at[pl.ds(base_t + ob, out_rows_per_tile)])

    out_type = [jax.ShapeDtypeStruct((tpad, d), jnp.float32)]
    scratch = [
        pltpu.VMEM((128,), jnp.int32),
        pltpu.VMEM((128, d), jnp.float32),
        pltpu.VMEM((128, d), jnp.float32),
    ]
    if with_counts:
        out_type.append(jax.ShapeDtypeStruct((tpad, 16), jnp.float32))
        scratch.append(pltpu.VMEM((128, 16), jnp.float32))
        scratch.append(pltpu.VMEM((128, 16), jnp.float32))
        scratch.append(pltpu.VMEM_SHARED((acc_rows, d), jnp.float32))
        scratch.append(pltpu.VMEM_SHARED((acc_rows, 16), jnp.float32))
    else:
        scratch.append(pltpu.VMEM_SHARED((acc_rows, d), jnp.float32))

    zeros = jnp.zeros((128, d), jnp.float32)
    zeros16 = jnp.zeros((128, 16), jnp.float32)
    ones = jnp.ones((128, 16), jnp.float32)
    res = pl.kernel(
        body,
        out_type=tuple(out_type),
        mesh=_sc_mesh(),
        compiler_params=pltpu.CompilerParams(use_tc_tiling_on_sc=False),
        scratch_types=scratch,
    )(vals, idx, zeros, zeros16, ones)
    return res if with_counts else res[0]


def _make_msg_body(i_dim, o_dim):
    def body(xj_ref, ea_ref, w1_ref, b1_ref, w2_ref, b2_ref, out_ref):
        xj = xj_ref[...]
        h = jnp.maximum(
            jnp.dot(ea_ref[...], w1_ref[...],
                    preferred_element_type=jnp.float32) + b1_ref[...], 0.0)
        W = jnp.dot(h, w2_ref[...],
                    preferred_element_type=jnp.float32) + b2_ref[...]
        blk = xj.shape[0]
        acc = jnp.zeros((blk, 128), jnp.float32)
        for i in range(i_dim):
            acc = acc + xj[:, i:i + 1] * W[:, i * 128:(i + 1) * 128]
        out_ref[...] = acc[:, :o_dim]
    return body


@functools.partial(jax.jit, static_argnames=("blk", "o"))
def _tc_msg(xj, ea8, w18, b1, w2p, b2p, blk, o):
    ep, i = xj.shape
    grid = ep // blk
    return pl.pallas_call(
        _make_msg_body(i, o),
        grid=(grid,),
        in_specs=[
            pl.BlockSpec((blk, i), lambda b: (b, 0)),
            pl.BlockSpec((blk, 8), lambda b: (b, 0)),
            pl.BlockSpec((8, 128), lambda b: (0, 0)),
            pl.BlockSpec((1, 128), lambda b: (0, 0)),
            pl.BlockSpec((128, i * 128), lambda b: (0, 0)),
            pl.BlockSpec((1, i * 128), lambda b: (0, 0)),
        ],
        out_specs=pl.BlockSpec((blk, o), lambda b: (b, 0)),
        out_shape=jax.ShapeDtypeStruct((ep, o), jnp.float32),
    )(xj, ea8, w18, b1, w2p, b2p)


def _elu(z):
    return jnp.where(z > 0.0, z, jnp.exp(jnp.minimum(z, 0.0)) - 1.0)


def _combine2_body(a1_ref, a2_ref, x_ref, root_ref, bias_ref, out_ref):
    z = (a1_ref[...] + a2_ref[...]
         + jnp.dot(x_ref[...], root_ref[...],
                   preferred_element_type=jnp.float32) + bias_ref[...])
    out_ref[...] = _elu(z)


@functools.partial(jax.jit, static_argnames=("blk",))
def _tc_combine2(a1, a2, x, root, bias2, blk):
    npad, o = a1.shape
    i = x.shape[1]
    return pl.pallas_call(
        _combine2_body,
        grid=(npad // blk,),
        in_specs=[
            pl.BlockSpec((blk, o), lambda b: (b, 0)),
            pl.BlockSpec((blk, o), lambda b: (b, 0)),
            pl.BlockSpec((blk, i), lambda b: (b, 0)),
            pl.BlockSpec((i, o), lambda b: (0, 0)),
            pl.BlockSpec((1, o), lambda b: (0, 0)),
        ],
        out_specs=pl.BlockSpec((blk, o), lambda b: (b, 0)),
        out_shape=jax.ShapeDtypeStruct((npad, o), jnp.float32),
    )(a1, a2, x, root, bias2)


def _combine_body(agg_ref, x_ref, root_ref, bias_ref, out_ref):
    z = (agg_ref[...]
         + jnp.dot(x_ref[...], root_ref[...],
                   preferred_element_type=jnp.float32) + bias_ref[...])
    out_ref[...] = _elu(z)


@functools.partial(jax.jit, static_argnames=("blk",))
def _tc_combine(agg, x, root, bias2, blk):
    npad, o = agg.shape
    i = x.shape[1]
    return pl.pallas_call(
        _combine_body,
        grid=(npad // blk,),
        in_specs=[
            pl.BlockSpec((blk, o), lambda b: (b, 0)),
            pl.BlockSpec((blk, i), lambda b: (b, 0)),
            pl.BlockSpec((i, o), lambda b: (0, 0)),
            pl.BlockSpec((1, o), lambda b: (0, 0)),
        ],
        out_specs=pl.BlockSpec((blk, o), lambda b: (b, 0)),
        out_shape=jax.ShapeDtypeStruct((npad, o), jnp.float32),
    )(agg, x, root, bias2)


def _div_body(s_ref, c_ref, out_ref):
    cnt = jnp.maximum(c_ref[:, 0:1], 1.0)
    out_ref[...] = s_ref[...] / cnt


@jax.jit
def _tc_divide(sums, counts):
    spad, d = sums.shape
    blk = 1024
    return pl.pallas_call(
        _div_body,
        grid=(spad // blk,),
        in_specs=[
            pl.BlockSpec((blk, d), lambda b: (b, 0)),
            pl.BlockSpec((blk, 16), lambda b: (b, 0)),
        ],
        out_specs=pl.BlockSpec((blk, d), lambda b: (b, 0)),
        out_shape=jax.ShapeDtypeStruct((spad, d), jnp.float32),
    )(sums, counts)


def _head_body(seg_ref, x_ref, f1w_ref, f1b_ref, f2w_ref, f2b_ref,
               f3w_ref, f3b_ref, out_ref, acc_s, acc_c):
    pid = pl.program_id(0)
    nsteps = pl.num_programs(0)

    @pl.when(pid == 0)
    def _():
        acc_s[...] = jnp.zeros_like(acc_s)
        acc_c[...] = jnp.zeros_like(acc_c)

    seg = seg_ref[0, 0, :]
    blk = seg.shape[0]
    seg2 = jnp.broadcast_to(seg[None, :], (G, blk))
    gids = lax.broadcasted_iota(jnp.int32, (G, blk), 0)
    oh = (seg2 == gids).astype(jnp.float32)
    xb = x_ref[...]
    acc_s[...] += jnp.dot(oh, xb, preferred_element_type=jnp.float32)
    acc_c[...] += jnp.dot(oh, jnp.ones_like(xb),
                          preferred_element_type=jnp.float32)

    @pl.when(pid == nsteps - 1)
    def _():
        mean = acc_s[...] / jnp.maximum(acc_c[...], 1.0)
        z1 = _elu(jnp.dot(mean, f1w_ref[...],
                          preferred_element_type=jnp.float32) + f1b_ref[...])
        z2 = _elu(jnp.dot(z1, f2w_ref[...],
                          preferred_element_type=jnp.float32) + f2b_ref[...])
        out_ref[...] = (jnp.dot(z2, f3w_ref[...],
                                preferred_element_type=jnp.float32)
                        + f3b_ref[...])


@jax.jit
def _tc_head(seg3, x2, f1w, f1b, f2w, f2b, f3w8, f3b8):
    blk = 1024
    nblk = seg3.shape[0]
    return pl.pallas_call(
        _head_body,
        grid=(nblk,),
        in_specs=[
            pl.BlockSpec((1, 1, blk), lambda b: (b, 0, 0)),
            pl.BlockSpec((blk, 64), lambda b: (b, 0)),
            pl.BlockSpec((64, 32), lambda b: (0, 0)),
            pl.BlockSpec((1, 32), lambda b: (0, 0)),
            pl.BlockSpec((32, 16), lambda b: (0, 0)),
            pl.BlockSpec((1, 16), lambda b: (0, 0)),
            pl.BlockSpec((16, 8), lambda b: (0, 0)),
            pl.BlockSpec((1, 8), lambda b: (0, 0)),
        ],
        out_specs=pl.BlockSpec((G, 8), lambda b: (0, 0)),
        out_shape=jax.ShapeDtypeStruct((G, 8), jnp.float32),
        scratch_shapes=[
            pltpu.VMEM((G, 64), jnp.float32),
            pltpu.VMEM((G, 64), jnp.float32),
        ],
    )(seg3, x2, f1w, f1b, f2w, f2b, f3w8, f3b8)


def _nnconv_layer(x, src, dst, ea8, w1, b1, w2, b2, root, bias, tpad, blk):
    i, o = root.shape
    w18 = jnp.zeros((8, 128), jnp.float32).at[:5].set(w1)
    b1r = b1.reshape(1, 128)
    w2p = jnp.zeros((128, i, 128), jnp.float32)
    w2p = w2p.at[:, :, :o].set(w2.reshape(128, i, o)).reshape(128, i * 128)
    b2p = jnp.zeros((i, 128), jnp.float32)
    b2p = b2p.at[:, :o].set(b2.reshape(i, o)).reshape(1, i * 128)
    ep = src.shape[0]
    half = ep // 2
    aggs = []
    for lo in (0, half):
        srch = lax.slice_in_dim(src, lo, lo + half)
        dsth = lax.slice_in_dim(dst, lo, lo + half)
        eah = lax.slice_in_dim(ea8, lo, lo + half)
        xjh = _sc_gather(x, srch, half // 32)
        msgh = _tc_msg(xjh, eah, w18, b1r, w2p, b2p, blk, o)
        aggs.append(_sc_scatter_add(msgh, dsth, tpad, False))
    return _tc_combine2(aggs[0], aggs[1], x, root, bias.reshape(1, o), 512)


def kernel(x, edge_index, edge_attr, node_to_subgraph, original_edge_index,
           original_edge_attr, subgraph_to_graph,
           sub0_nw1, sub0_nb1, sub0_nw2, sub0_nb2, sub0_root, sub0_bias,
           sub1_nw1, sub1_nb1, sub1_nw2, sub1_nb2, sub1_root, sub1_bias,
           sub2_nw1, sub2_nb1, sub2_nw2, sub2_nb2, sub2_root, sub2_bias,
           gl0_nw1, gl0_nb1, gl0_nw2, gl0_nb2, gl0_root, gl0_bias,
           gl1_nw1, gl1_nb1, gl1_nw2, gl1_nb2, gl1_root, gl1_bias,
           fc1_w, fc1_b, fc2_w, fc2_b, fc3_w, fc3_b):
    f32 = jnp.float32
    x = x.astype(f32)

    src = jnp.pad(edge_index[0].astype(jnp.int32), (0, EP1 - E))
    dst = jnp.pad(edge_index[1].astype(jnp.int32), (0, EP1 - E),
                  constant_values=NPAD)
    ea8 = jnp.zeros((EP1, 8), f32).at[:E, :5].set(edge_attr)
    osrc = jnp.pad(original_edge_index[0].astype(jnp.int32), (0, EP2 - EO))
    odst = jnp.pad(original_edge_index[1].astype(jnp.int32), (0, EP2 - EO),
                   constant_values=SPAD)
    oea8 = jnp.zeros((EP2, 8), f32).at[:EO, :5].set(original_edge_attr)
    n2s = jnp.pad(node_to_subgraph.astype(jnp.int32), (0, NPAD - N),
                  constant_values=SPAD)
    s2g = jnp.pad(subgraph_to_graph.astype(jnp.int32), (0, SPAD - S),
                  constant_values=G)

    xp = jnp.zeros((NPAD, F_IN), f32).at[:N].set(x)

    subs = [
        (sub0_nw1, sub0_nb1, sub0_nw2, sub0_nb2, sub0_root, sub0_bias),
        (sub1_nw1, sub1_nb1, sub1_nw2, sub1_nb2, sub1_root, sub1_bias),
        (sub2_nw1, sub2_nb1, sub2_nw2, sub2_nb2, sub2_root, sub2_bias),
    ]
    h = xp
    for (w1, b1, w2, b2, root, bias) in subs:
        h = _nnconv_layer(h, src, dst, ea8, w1, b1, w2, b2, root, bias,
                          NPAD, 256)

    sums, counts = _sc_scatter_add(h, n2s, SPAD, True)
    p = _tc_divide(sums, counts)

    for (w1, b1, w2, b2, root, bias) in [
            (gl0_nw1, gl0_nb1, gl0_nw2, gl0_nb2, gl0_root, gl0_bias),
            (gl1_nw1, gl1_nb1, gl1_nw2, gl1_nb2, gl1_root, gl1_bias)]:
        p = _nnconv_layer(p, osrc, odst, oea8, w1, b1, w2, b2, root, bias,
                          SPAD, 256)

    seg3 = s2g.reshape(SPAD // 1024, 1, 1024)
    f3w8 = jnp.zeros((16, 8), f32).at[:, 0:1].set(fc3_w)
    f3b8 = jnp.zeros((1, 8), f32).at[0, 0].set(fc3_b[0])
    out8 = _tc_head(seg3, p, fc1_w, fc1_b.reshape(1, 32),
                    fc2_w, fc2_b.reshape(1, 16), f3w8, f3b8)
    return out8[:, 0]


# speedup vs baseline: 1.2671x; 1.0126x over previous
import functools
import jax
import jax.numpy as jnp
from jax import lax
from jax.experimental import pallas as pl
from jax.experimental.pallas import tpu as pltpu
from jax.experimental.pallas import tpu_sc as plsc

N = 50000
F_IN = 16
E = 40000
S = 10000
EO = 20000
G = 256

NPAD = 51200
SPAD = 10240
EP1 = 40960
EP2 = 20480

@functools.cache
def _sc_mesh():
    return plsc.VectorSubcoreMesh(
        core_axis_name="c", subcore_axis_name="s",
        num_cores=2, num_subcores=16)


@functools.partial(jax.jit, static_argnames=("rows_per_worker",))
def _sc_gather(tab, idx, rows_per_worker):
    ep, d = idx.shape[0], tab.shape[1]
    chunk = 128 if rows_per_worker % 128 == 0 else 64
    nchunks = rows_per_worker // chunk

    def body(tab_ref, idx_ref, out_ref, idx_all, rows_v0, rows_v1,
             gsem0, gsem1, wsem0, wsem1):
        wid = lax.axis_index("s") * 2 + lax.axis_index("c")
        base = wid * rows_per_worker
        rows = (rows_v0, rows_v1)
        gsem = (gsem0, gsem1)
        wsem = (wsem0, wsem1)
        pltpu.sync_copy(idx_ref.at[pl.ds(base, rows_per_worker)], idx_all)
        gd = [None, None]
        wd = [None, None]
        for c in range(nchunks):
            p = c % 2
            if c >= 2:
                wd[p].wait()
            gd[p] = pltpu.async_copy(
                tab_ref.at[idx_all.at[pl.ds(c * chunk, chunk)]],
                rows[p], gsem[p])
            if c >= 1:
                q = 1 - p
                gd[q].wait()
                wd[q] = pltpu.async_copy(
                    rows[q], out_ref.at[pl.ds(base + (c - 1) * chunk, chunk)],
                    wsem[q])
        lp = (nchunks - 1) % 2
        gd[lp].wait()
        wd[lp] = pltpu.async_copy(
            rows[lp], out_ref.at[pl.ds(base + (nchunks - 1) * chunk, chunk)],
            wsem[lp])
        if nchunks >= 2:
            wd[1 - lp].wait()
        wd[lp].wait()

    return pl.kernel(
        body,
        out_type=jax.ShapeDtypeStruct((ep, d), jnp.float32),
        mesh=_sc_mesh(),
        compiler_params=pltpu.CompilerParams(use_tc_tiling_on_sc=False),
        scratch_types=[
            pltpu.VMEM((rows_per_worker,), jnp.int32),
            pltpu.VMEM((chunk, d), jnp.float32),
            pltpu.VMEM((chunk, d), jnp.float32),
            pltpu.SemaphoreType.DMA,
            pltpu.SemaphoreType.DMA,
            pltpu.SemaphoreType.DMA,
            pltpu.SemaphoreType.DMA,
        ],
    )(tab, idx)


@functools.partial(jax.jit, static_argnames=("tpad", "with_counts"))
def _sc_scatter_add(vals, idx, tpad, with_counts):
    rp, d = vals.shape
    t2 = tpad // 2
    acc_rows = ((t2 + 1 + 127) // 128) * 128
    nzc = acc_rows // 128
    zc_per_tile = (nzc + 15) // 16
    rows_per_tile = rp // 16
    in_chunks = rows_per_tile // 128
    out_rows_per_tile = t2 // 16

    def body(vals_ref, idx_ref, zero_ref, z16_ref, one_ref, *refs):
        if with_counts:
            (out_ref, cnt_ref, idx_v0, idx_v1, rows_v0, rows_v1,
             zero_v, z16_v, ones_v,
             isem0, isem1, vsem0, vsem1, ssem0, ssem1, csem0, csem1,
             acc, accc) = refs
            csem = (csem0, csem1)
        else:
            (out_ref, idx_v0, idx_v1, rows_v0, rows_v1, zero_v,
             isem0, isem1, vsem0, vsem1, ssem0, ssem1, acc) = refs
        idx_v = (idx_v0, idx_v1)
        rows_v = (rows_v0, rows_v1)
        isem = (isem0, isem1)
        vsem = (vsem0, vsem1)
        ssem = (ssem0, ssem1)
        c = lax.axis_index("c")
        sid = lax.axis_index("s")
        base_t = c * t2

        pltpu.sync_copy(zero_ref, zero_v)
        if with_counts:
            pltpu.sync_copy(one_ref, ones_v)
            pltpu.sync_copy(z16_ref, z16_v)

        def zstep(j, carry):
            ch = sid * zc_per_tile + j

            @pl.when(ch < nzc)
            def _():
                pltpu.sync_copy(zero_v, acc.at[pl.ds(ch * 128, 128)])
                if with_counts:
                    pltpu.sync_copy(z16_v, accc.at[pl.ds(ch * 128, 128)])
            return carry

        lax.fori_loop(0, zc_per_tile, zstep, 0, unroll=False)
        plsc.subcore_barrier()

        dI = [None, None]
        dV = [None, None]
        dS = [None, None]
        dC = [None, None]
        for c in range(in_chunks):
            p = c % 2
            if c >= 2:
                dS[p].wait()
                if with_counts:
                    dC[p].wait()
            off = sid * rows_per_tile + c * 128
            dI[p] = pltpu.async_copy(idx_ref.at[pl.ds(off, 128)],
                                     idx_v[p], isem[p])
            dV[p] = pltpu.async_copy(vals_ref.at[pl.ds(off, 128)],
                                     rows_v[p], vsem[p])
            dI[p].wait()
            for j in range(8):
                v = idx_v[p][pl.ds(j * 16, 16)]
                loc = v - base_t
                bad = (loc < 0) | (loc >= t2)
                idx_v[p][pl.ds(j * 16, 16)] = jnp.where(bad, t2, loc)
            dV[p].wait()
            dS[p] = pltpu.async_copy(rows_v[p], acc.at[idx_v[p]],
                                     ssem[p], add=True)
            if with_counts:
                dC[p] = pltpu.async_copy(ones_v, accc.at[idx_v[p]],
                                         csem[p], add=True)
        for p in (0, 1):
            if dS[p] is not None:
                dS[p].wait()
            if with_counts and dC[p] is not None:
                dC[p].wait()
        plsc.subcore_barrier()

        ob = sid * out_rows_per_tile
        pltpu.sync_copy(acc.at[pl.ds(ob, out_rows_per_tile)],
                        out_ref.at[pl.ds(base_t + ob, out_rows_per_tile)])
        if with_counts:
            pltpu.sync_copy(accc.at[pl.ds(ob, out_rows_per_tile)],
                            cnt_ref.at[pl.ds(base_t + ob, out_rows_per_tile)])

    out_type = [jax.ShapeDtypeStruct((tpad, d), jnp.float32)]
    scratch = [
        pltpu.VMEM((128,), jnp.int32),
        pltpu.VMEM((128,), jnp.int32),
        pltpu.VMEM((128, d), jnp.float32),
        pltpu.VMEM((128, d), jnp.float32),
        pltpu.VMEM((128, d), jnp.float32),
    ]
    if with_counts:
        out_type.append(jax.ShapeDtypeStruct((tpad, 16), jnp.float32))
        scratch.append(pltpu.VMEM((128, 16), jnp.float32))
        scratch.append(pltpu.VMEM((128, 16), jnp.float32))
        scratch.extend([pltpu.SemaphoreType.DMA] * 8)
        scratch.append(pltpu.VMEM_SHARED((acc_rows, d), jnp.float32))
        scratch.append(pltpu.VMEM_SHARED((acc_rows, 16), jnp.float32))
    else:
        scratch.extend([pltpu.SemaphoreType.DMA] * 6)
        scratch.append(pltpu.VMEM_SHARED((acc_rows, d), jnp.float32))

    zeros = jnp.zeros((128, d), jnp.float32)
    zeros16 = jnp.zeros((128, 16), jnp.float32)
    ones = jnp.ones((128, 16), jnp.float32)
    res = pl.kernel(
        body,
        out_type=tuple(out_type),
        mesh=_sc_mesh(),
        compiler_params=pltpu.CompilerParams(use_tc_tiling_on_sc=False),
        scratch_types=scratch,
    )(vals, idx, zeros, zeros16, ones)
    return res if with_counts else res[0]


def _make_msg_body(i_dim, o_dim):
    def body(xj_ref, ea_ref, w1_ref, b1_ref, w2_ref, b2_ref, out_ref):
        xj = xj_ref[...]
        h = jnp.maximum(
            jnp.dot(ea_ref[...], w1_ref[...],
                    preferred_element_type=jnp.float32) + b1_ref[...], 0.0)
        W = jnp.dot(h, w2_ref[...],
                    preferred_element_type=jnp.float32) + b2_ref[...]
        blk = xj.shape[0]
        acc = jnp.zeros((blk, 128), jnp.float32)
        for i in range(i_dim):
            acc = acc + xj[:, i:i + 1] * W[:, i * 128:(i + 1) * 128]
        out_ref[...] = acc[:, :o_dim]
    return body


@functools.partial(jax.jit, static_argnames=("blk", "o"))
def _tc_msg(xj, ea8, w18, b1, w2p, b2p, blk, o):
    ep, i = xj.shape
    grid = ep // blk
    return pl.pallas_call(
        _make_msg_body(i, o),
        grid=(grid,),
        in_specs=[
            pl.BlockSpec((blk, i), lambda b: (b, 0)),
            pl.BlockSpec((blk, 8), lambda b: (b, 0)),
            pl.BlockSpec((8, 128), lambda b: (0, 0)),
            pl.BlockSpec((1, 128), lambda b: (0, 0)),
            pl.BlockSpec((128, i * 128), lambda b: (0, 0)),
            pl.BlockSpec((1, i * 128), lambda b: (0, 0)),
        ],
        out_specs=pl.BlockSpec((blk, o), lambda b: (b, 0)),
        out_shape=jax.ShapeDtypeStruct((ep, o), jnp.float32),
    )(xj, ea8, w18, b1, w2p, b2p)


def _elu(z):
    return jnp.where(z > 0.0, z, jnp.exp(jnp.minimum(z, 0.0)) - 1.0)


def _combine2_body(a1_ref, a2_ref, x_ref, root_ref, bias_ref, out_ref):
    z = (a1_ref[...] + a2_ref[...]
         + jnp.dot(x_ref[...], root_ref[...],
                   preferred_element_type=jnp.float32) + bias_ref[...])
    out_ref[...] = _elu(z)


@functools.partial(jax.jit, static_argnames=("blk",))
def _tc_combine2(a1, a2, x, root, bias2, blk):
    npad, o = a1.shape
    i = x.shape[1]
    return pl.pallas_call(
        _combine2_body,
        grid=(npad // blk,),
        in_specs=[
            pl.BlockSpec((blk, o), lambda b: (b, 0)),
            pl.BlockSpec((blk, o), lambda b: (b, 0)),
            pl.BlockSpec((blk, i), lambda b: (b, 0)),
            pl.BlockSpec((i, o), lambda b: (0, 0)),
            pl.BlockSpec((1, o), lambda b: (0, 0)),
        ],
        out_specs=pl.BlockSpec((blk, o), lambda b: (b, 0)),
        out_shape=jax.ShapeDtypeStruct((npad, o), jnp.float32),
    )(a1, a2, x, root, bias2)


def _combine_body(agg_ref, x_ref, root_ref, bias_ref, out_ref):
    z = (agg_ref[...]
         + jnp.dot(x_ref[...], root_ref[...],
                   preferred_element_type=jnp.float32) + bias_ref[...])
    out_ref[...] = _elu(z)


@functools.partial(jax.jit, static_argnames=("blk",))
def _tc_combine(agg, x, root, bias2, blk):
    npad, o = agg.shape
    i = x.shape[1]
    return pl.pallas_call(
        _combine_body,
        grid=(npad // blk,),
        in_specs=[
            pl.BlockSpec((blk, o), lambda b: (b, 0)),
            pl.BlockSpec((blk, i), lambda b: (b, 0)),
            pl.BlockSpec((i, o), lambda b: (0, 0)),
            pl.BlockSpec((1, o), lambda b: (0, 0)),
        ],
        out_specs=pl.BlockSpec((blk, o), lambda b: (b, 0)),
        out_shape=jax.ShapeDtypeStruct((npad, o), jnp.float32),
    )(agg, x, root, bias2)


def _div_body(s_ref, c_ref, out_ref):
    cnt = jnp.maximum(c_ref[:, 0:1], 1.0)
    out_ref[...] = s_ref[...] / cnt


@jax.jit
def _tc_divide(sums, counts):
    spad, d = sums.shape
    blk = 1024
    return pl.pallas_call(
        _div_body,
        grid=(spad // blk,),
        in_specs=[
            pl.BlockSpec((blk, d), lambda b: (b, 0)),
            pl.BlockSpec((blk, 16), lambda b: (b, 0)),
        ],
        out_specs=pl.BlockSpec((blk, d), lambda b: (b, 0)),
        out_shape=jax.ShapeDtypeStruct((spad, d), jnp.float32),
    )(sums, counts)


def _head_body(seg_ref, x_ref, f1w_ref, f1b_ref, f2w_ref, f2b_ref,
               f3w_ref, f3b_ref, out_ref, acc_s, acc_c):
    pid = pl.program_id(0)
    nsteps = pl.num_programs(0)

    @pl.when(pid == 0)
    def _():
        acc_s[...] = jnp.zeros_like(acc_s)
        acc_c[...] = jnp.zeros_like(acc_c)

    seg = seg_ref[0, 0, :]
    blk = seg.shape[0]
    seg2 = jnp.broadcast_to(seg[None, :], (G, blk))
    gids = lax.broadcasted_iota(jnp.int32, (G, blk), 0)
    oh = (seg2 == gids).astype(jnp.float32)
    xb = x_ref[...]
    acc_s[...] += jnp.dot(oh, xb, preferred_element_type=jnp.float32)
    acc_c[...] += jnp.dot(oh, jnp.ones_like(xb),
                          preferred_element_type=jnp.float32)

    @pl.when(pid == nsteps - 1)
    def _():
        mean = acc_s[...] / jnp.maximum(acc_c[...], 1.0)
        z1 = _elu(jnp.dot(mean, f1w_ref[...],
                          preferred_element_type=jnp.float32) + f1b_ref[...])
        z2 = _elu(jnp.dot(z1, f2w_ref[...],
                          preferred_element_type=jnp.float32) + f2b_ref[...])
        out_ref[...] = (jnp.dot(z2, f3w_ref[...],
                                preferred_element_type=jnp.float32)
                        + f3b_ref[...])


@jax.jit
def _tc_head(seg3, x2, f1w, f1b, f2w, f2b, f3w8, f3b8):
    blk = 1024
    nblk = seg3.shape[0]
    return pl.pallas_call(
        _head_body,
        grid=(nblk,),
        in_specs=[
            pl.BlockSpec((1, 1, blk), lambda b: (b, 0, 0)),
            pl.BlockSpec((blk, 64), lambda b: (b, 0)),
            pl.BlockSpec((64, 32), lambda b: (0, 0)),
            pl.BlockSpec((1, 32), lambda b: (0, 0)),
            pl.BlockSpec((32, 16), lambda b: (0, 0)),
            pl.BlockSpec((1, 16), lambda b: (0, 0)),
            pl.BlockSpec((16, 8), lambda b: (0, 0)),
            pl.BlockSpec((1, 8), lambda b: (0, 0)),
        ],
        out_specs=pl.BlockSpec((G, 8), lambda b: (0, 0)),
        out_shape=jax.ShapeDtypeStruct((G, 8), jnp.float32),
        scratch_shapes=[
            pltpu.VMEM((G, 64), jnp.float32),
            pltpu.VMEM((G, 64), jnp.float32),
        ],
    )(seg3, x2, f1w, f1b, f2w, f2b, f3w8, f3b8)


def _nnconv_layer(x, src, dst, ea8, w1, b1, w2, b2, root, bias, tpad, blk):
    i, o = root.shape
    w18 = jnp.zeros((8, 128), jnp.float32).at[:5].set(w1)
    b1r = b1.reshape(1, 128)
    w2p = jnp.zeros((128, i, 128), jnp.float32)
    w2p = w2p.at[:, :, :o].set(w2.reshape(128, i, o)).reshape(128, i * 128)
    b2p = jnp.zeros((i, 128), jnp.float32)
    b2p = b2p.at[:, :o].set(b2.reshape(i, o)).reshape(1, i * 128)
    ep = src.shape[0]
    half = ep // 2
    aggs = []
    for lo in (0, half):
        srch = lax.slice_in_dim(src, lo, lo + half)
        dsth = lax.slice_in_dim(dst, lo, lo + half)
        eah = lax.slice_in_dim(ea8, lo, lo + half)
        xjh = _sc_gather(x, srch, half // 32)
        msgh = _tc_msg(xjh, eah, w18, b1r, w2p, b2p, blk, o)
        aggs.append(_sc_scatter_add(msgh, dsth, tpad, False))
    return _tc_combine2(aggs[0], aggs[1], x, root, bias.reshape(1, o), 512)


def kernel(x, edge_index, edge_attr, node_to_subgraph, original_edge_index,
           original_edge_attr, subgraph_to_graph,
           sub0_nw1, sub0_nb1, sub0_nw2, sub0_nb2, sub0_root, sub0_bias,
           sub1_nw1, sub1_nb1, sub1_nw2, sub1_nb2, sub1_root, sub1_bias,
           sub2_nw1, sub2_nb1, sub2_nw2, sub2_nb2, sub2_root, sub2_bias,
           gl0_nw1, gl0_nb1, gl0_nw2, gl0_nb2, gl0_root, gl0_bias,
           gl1_nw1, gl1_nb1, gl1_nw2, gl1_nb2, gl1_root, gl1_bias,
           fc1_w, fc1_b, fc2_w, fc2_b, fc3_w, fc3_b):
    f32 = jnp.float32
    x = x.astype(f32)

    src = jnp.pad(edge_index[0].astype(jnp.int32), (0, EP1 - E))
    dst = jnp.pad(edge_index[1].astype(jnp.int32), (0, EP1 - E),
                  constant_values=NPAD)
    ea8 = jnp.zeros((EP1, 8), f32).at[:E, :5].set(edge_attr)
    osrc = jnp.pad(original_edge_index[0].astype(jnp.int32), (0, EP2 - EO))
    odst = jnp.pad(original_edge_index[1].astype(jnp.int32), (0, EP2 - EO),
                   constant_values=SPAD)
    oea8 = jnp.zeros((EP2, 8), f32).at[:EO, :5].set(original_edge_attr)
    n2s = jnp.pad(node_to_subgraph.astype(jnp.int32), (0, NPAD - N),
                  constant_values=SPAD)
    s2g = jnp.pad(subgraph_to_graph.astype(jnp.int32), (0, SPAD - S),
                  constant_values=G)

    xp = jnp.zeros((NPAD, F_IN), f32).at[:N].set(x)

    subs = [
        (sub0_nw1, sub0_nb1, sub0_nw2, sub0_nb2, sub0_root, sub0_bias),
        (sub1_nw1, sub1_nb1, sub1_nw2, sub1_nb2, sub1_root, sub1_bias),
        (sub2_nw1, sub2_nb1, sub2_nw2, sub2_nb2, sub2_root, sub2_bias),
    ]
    h = xp
    for (w1, b1, w2, b2, root, bias) in subs:
        h = _nnconv_layer(h, src, dst, ea8, w1, b1, w2, b2, root, bias,
                          NPAD, 256)

    sums, counts = _sc_scatter_add(h, n2s, SPAD, True)
    p = _tc_divide(sums, counts)

    for (w1, b1, w2, b2, root, bias) in [
            (gl0_nw1, gl0_nb1, gl0_nw2, gl0_nb2, gl0_root, gl0_bias),
            (gl1_nw1, gl1_nb1, gl1_nw2, gl1_nb2, gl1_root, gl1_bias)]:
        p = _nnconv_layer(p, osrc, odst, oea8, w1, b1, w2, b2, root, bias,
                          SPAD, 256)

    seg3 = s2g.reshape(SPAD // 1024, 1, 1024)
    f3w8 = jnp.zeros((16, 8), f32).at[:, 0:1].set(fc3_w)
    f3b8 = jnp.zeros((1, 8), f32).at[0, 0].set(fc3_b[0])
    out8 = _tc_head(seg3, p, fc1_w, fc1_b.reshape(1, 32),
                    fc2_w, fc2_b.reshape(1, 16), f3w8, f3b8)
    return out8[:, 0]


# speedup vs baseline: 1.2928x; 1.0203x over previous
import functools
import jax
import jax.numpy as jnp
from jax import lax
from jax.experimental import pallas as pl
from jax.experimental.pallas import tpu as pltpu
from jax.experimental.pallas import tpu_sc as plsc

N = 50000
F_IN = 16
E = 40000
S = 10000
EO = 20000
G = 256

NPAD = 51200
SPAD = 10240
EP1 = 40960
EP2 = 20480

@functools.cache
def _sc_mesh():
    return plsc.VectorSubcoreMesh(
        core_axis_name="c", subcore_axis_name="s",
        num_cores=2, num_subcores=16)


@functools.partial(jax.jit, static_argnames=("rows_per_worker",))
def _sc_gather(tab, idx, rows_per_worker):
    ep, d = idx.shape[0], tab.shape[1]
    chunk = 128 if rows_per_worker % 128 == 0 else 64
    nchunks = rows_per_worker // chunk

    def body(tab_ref, idx_ref, out_ref, idx_all, rows_v0, rows_v1,
             gsem0, gsem1, wsem0, wsem1):
        wid = lax.axis_index("s") * 2 + lax.axis_index("c")
        base = wid * rows_per_worker
        rows = (rows_v0, rows_v1)
        gsem = (gsem0, gsem1)
        wsem = (wsem0, wsem1)
        pltpu.sync_copy(idx_ref.at[pl.ds(base, rows_per_worker)], idx_all)
        gd = [None, None]
        wd = [None, None]
        for c in range(nchunks):
            p = c % 2
            if c >= 2:
                wd[p].wait()
            gd[p] = pltpu.async_copy(
                tab_ref.at[idx_all.at[pl.ds(c * chunk, chunk)]],
                rows[p], gsem[p])
            if c >= 1:
                q = 1 - p
                gd[q].wait()
                wd[q] = pltpu.async_copy(
                    rows[q], out_ref.at[pl.ds(base + (c - 1) * chunk, chunk)],
                    wsem[q])
        lp = (nchunks - 1) % 2
        gd[lp].wait()
        wd[lp] = pltpu.async_copy(
            rows[lp], out_ref.at[pl.ds(base + (nchunks - 1) * chunk, chunk)],
            wsem[lp])
        if nchunks >= 2:
            wd[1 - lp].wait()
        wd[lp].wait()

    return pl.kernel(
        body,
        out_type=jax.ShapeDtypeStruct((ep, d), jnp.float32),
        mesh=_sc_mesh(),
        compiler_params=pltpu.CompilerParams(use_tc_tiling_on_sc=False),
        scratch_types=[
            pltpu.VMEM((rows_per_worker,), jnp.int32),
            pltpu.VMEM((chunk, d), jnp.float32),
            pltpu.VMEM((chunk, d), jnp.float32),
            pltpu.SemaphoreType.DMA,
            pltpu.SemaphoreType.DMA,
            pltpu.SemaphoreType.DMA,
            pltpu.SemaphoreType.DMA,
        ],
    )(tab, idx)


@functools.partial(jax.jit, static_argnames=("tpad", "with_counts"))
def _sc_scatter_add(vals, idx, tpad, with_counts):
    rp, d = vals.shape
    t2 = tpad // 2
    acc_rows = ((t2 + 1 + 127) // 128) * 128
    nzc = acc_rows // 128
    zc_per_tile = (nzc + 15) // 16
    rows_per_tile = rp // 16
    in_chunks = rows_per_tile // 128
    out_rows_per_tile = t2 // 16

    def body(vals_ref, idx_ref, zero_ref, z16_ref, one_ref, *refs):
        if with_counts:
            (out_ref, cnt_ref, idx_v0, idx_v1, rows_v0, rows_v1,
             zero_v, z16_v, ones_v,
             isem0, isem1, vsem0, vsem1, ssem0, ssem1, csem0, csem1,
             acc, accc) = refs
            csem = (csem0, csem1)
        else:
            (out_ref, idx_v0, idx_v1, rows_v0, rows_v1, zero_v,
             isem0, isem1, vsem0, vsem1, ssem0, ssem1, acc) = refs
        idx_v = (idx_v0, idx_v1)
        rows_v = (rows_v0, rows_v1)
        isem = (isem0, isem1)
        vsem = (vsem0, vsem1)
        ssem = (ssem0, ssem1)
        c = lax.axis_index("c")
        sid = lax.axis_index("s")
        base_t = c * t2

        pltpu.sync_copy(zero_ref, zero_v)
        if with_counts:
            pltpu.sync_copy(one_ref, ones_v)
            pltpu.sync_copy(z16_ref, z16_v)

        def zstep(j, carry):
            ch = sid * zc_per_tile + j

            @pl.when(ch < nzc)
            def _():
                pltpu.sync_copy(zero_v, acc.at[pl.ds(ch * 128, 128)])
                if with_counts:
                    pltpu.sync_copy(z16_v, accc.at[pl.ds(ch * 128, 128)])
            return carry

        lax.fori_loop(0, zc_per_tile, zstep, 0, unroll=False)
        plsc.subcore_barrier()

        dI = [None, None]
        dV = [None, None]
        dS = [None, None]
        dC = [None, None]
        for c in range(in_chunks):
            p = c % 2
            if c >= 2:
                dS[p].wait()
                if with_counts:
                    dC[p].wait()
            off = sid * rows_per_tile + c * 128
            dI[p] = pltpu.async_copy(idx_ref.at[pl.ds(off, 128)],
                                     idx_v[p], isem[p])
            dV[p] = pltpu.async_copy(vals_ref.at[pl.ds(off, 128)],
                                     rows_v[p], vsem[p])
            dI[p].wait()
            for j in range(8):
                v = idx_v[p][pl.ds(j * 16, 16)]
                loc = v - base_t
                bad = (loc < 0) | (loc >= t2)
                idx_v[p][pl.ds(j * 16, 16)] = jnp.where(bad, t2, loc)
            dV[p].wait()
            dS[p] = pltpu.async_copy(rows_v[p], acc.at[idx_v[p]],
                                     ssem[p], add=True)
            if with_counts:
                dC[p] = pltpu.async_copy(ones_v, accc.at[idx_v[p]],
                                         csem[p], add=True)
        for p in (0, 1):
            if dS[p] is not None:
                dS[p].wait()
            if with_counts and dC[p] is not None:
                dC[p].wait()
        plsc.subcore_barrier()

        ob = sid * out_rows_per_tile
        pltpu.sync_copy(acc.at[pl.ds(ob, out_rows_per_tile)],
                        out_ref.at[pl.ds(base_t + ob, out_rows_per_tile)])
        if with_counts:
            pltpu.sync_copy(accc.at[pl.ds(ob, out_rows_per_tile)],
                            cnt_ref.at[pl.ds(base_t + ob, out_rows_per_tile)])

    out_type = [jax.ShapeDtypeStruct((tpad, d), jnp.float32)]
    scratch = [
        pltpu.VMEM((128,), jnp.int32),
        pltpu.VMEM((128,), jnp.int32),
        pltpu.VMEM((128, d), jnp.float32),
        pltpu.VMEM((128, d), jnp.float32),
        pltpu.VMEM((128, d), jnp.float32),
    ]
    if with_counts:
        out_type.append(jax.ShapeDtypeStruct((tpad, 16), jnp.float32))
        scratch.append(pltpu.VMEM((128, 16), jnp.float32))
        scratch.append(pltpu.VMEM((128, 16), jnp.float32))
        scratch.extend([pltpu.SemaphoreType.DMA] * 8)
        scratch.append(pltpu.VMEM_SHARED((acc_rows, d), jnp.float32))
        scratch.append(pltpu.VMEM_SHARED((acc_rows, 16), jnp.float32))
    else:
        scratch.extend([pltpu.SemaphoreType.DMA] * 6)
        scratch.append(pltpu.VMEM_SHARED((acc_rows, d), jnp.float32))

    zeros = jnp.zeros((128, d), jnp.float32)
    zeros16 = jnp.zeros((128, 16), jnp.float32)
    ones = jnp.ones((128, 16), jnp.float32)
    res = pl.kernel(
        body,
        out_type=tuple(out_type),
        mesh=_sc_mesh(),
        compiler_params=pltpu.CompilerParams(use_tc_tiling_on_sc=False),
        scratch_types=scratch,
    )(vals, idx, zeros, zeros16, ones)
    return res if with_counts else res[0]


def _make_msg_body(i_dim, o_dim):
    def body(xj_ref, ea_ref, w1_ref, b1_ref, w2_ref, b2_ref, out_ref):
        xj = xj_ref[...]
        h = jnp.maximum(
            jnp.dot(ea_ref[...], w1_ref[...],
                    preferred_element_type=jnp.float32) + b1_ref[...], 0.0)
        W = jnp.dot(h, w2_ref[...],
                    preferred_element_type=jnp.float32) + b2_ref[...]
        blk = xj.shape[0]
        xjr = xj.astype(jnp.bfloat16).astype(jnp.float32)
        Wr = W.astype(jnp.bfloat16).astype(jnp.float32)
        acc = jnp.zeros((blk, 128), jnp.float32)
        for i in range(i_dim):
            acc = acc + xjr[:, i:i + 1] * Wr[:, i * 128:(i + 1) * 128]
        out_ref[...] = acc[:, :o_dim]
    return body


@functools.partial(jax.jit, static_argnames=("blk", "o"))
def _tc_msg(xj, ea8, w18, b1, w2p, b2p, blk, o):
    ep, i = xj.shape
    grid = ep // blk
    return pl.pallas_call(
        _make_msg_body(i, o),
        grid=(grid,),
        in_specs=[
            pl.BlockSpec((blk, i), lambda b: (b, 0)),
            pl.BlockSpec((blk, 8), lambda b: (b, 0)),
            pl.BlockSpec((8, 128), lambda b: (0, 0)),
            pl.BlockSpec((1, 128), lambda b: (0, 0)),
            pl.BlockSpec((128, i * 128), lambda b: (0, 0)),
            pl.BlockSpec((1, i * 128), lambda b: (0, 0)),
        ],
        out_specs=pl.BlockSpec((blk, o), lambda b: (b, 0)),
        out_shape=jax.ShapeDtypeStruct((ep, o), jnp.float32),
    )(xj, ea8, w18, b1, w2p, b2p)


def _elu(z):
    return jnp.where(z > 0.0, z, jnp.exp(jnp.minimum(z, 0.0)) - 1.0)


def _combine2_body(a1_ref, a2_ref, x_ref, root_ref, bias_ref, out_ref):
    z = (a1_ref[...] + a2_ref[...]
         + jnp.dot(x_ref[...], root_ref[...],
                   preferred_element_type=jnp.float32) + bias_ref[...])
    out_ref[...] = _elu(z)


@functools.partial(jax.jit, static_argnames=("blk",))
def _tc_combine2(a1, a2, x, root, bias2, blk):
    npad, o = a1.shape
    i = x.shape[1]
    return pl.pallas_call(
        _combine2_body,
        grid=(npad // blk,),
        in_specs=[
            pl.BlockSpec((blk, o), lambda b: (b, 0)),
            pl.BlockSpec((blk, o), lambda b: (b, 0)),
            pl.BlockSpec((blk, i), lambda b: (b, 0)),
            pl.BlockSpec((i, o), lambda b: (0, 0)),
            pl.BlockSpec((1, o), lambda b: (0, 0)),
        ],
        out_specs=pl.BlockSpec((blk, o), lambda b: (b, 0)),
        out_shape=jax.ShapeDtypeStruct((npad, o), jnp.float32),
    )(a1, a2, x, root, bias2)


def _combine_body(agg_ref, x_ref, root_ref, bias_ref, out_ref):
    z = (agg_ref[...]
         + jnp.dot(x_ref[...], root_ref[...],
                   preferred_element_type=jnp.float32) + bias_ref[...])
    out_ref[...] = _elu(z)


@functools.partial(jax.jit, static_argnames=("blk",))
def _tc_combine(agg, x, root, bias2, blk):
    npad, o = agg.shape
    i = x.shape[1]
    return pl.pallas_call(
        _combine_body,
        grid=(npad // blk,),
        in_specs=[
            pl.BlockSpec((blk, o), lambda b: (b, 0)),
            pl.BlockSpec((blk, i), lambda b: (b, 0)),
            pl.BlockSpec((i, o), lambda b: (0, 0)),
            pl.BlockSpec((1, o), lambda b: (0, 0)),
        ],
        out_specs=pl.BlockSpec((blk, o), lambda b: (b, 0)),
        out_shape=jax.ShapeDtypeStruct((npad, o), jnp.float32),
    )(agg, x, root, bias2)


def _div_body(s_ref, c_ref, out_ref):
    cnt = jnp.maximum(c_ref[:, 0:1], 1.0)
    out_ref[...] = s_ref[...] / cnt


@jax.jit
def _tc_divide(sums, counts):
    spad, d = sums.shape
    blk = 1024
    return pl.pallas_call(
        _div_body,
        grid=(spad // blk,),
        in_specs=[
            pl.BlockSpec((blk, d), lambda b: (b, 0)),
            pl.BlockSpec((blk, 16), lambda b: (b, 0)),
        ],
        out_specs=pl.BlockSpec((blk, d), lambda b: (b, 0)),
        out_shape=jax.ShapeDtypeStruct((spad, d), jnp.float32),
    )(sums, counts)


def _head_body(seg_ref, x_ref, f1w_ref, f1b_ref, f2w_ref, f2b_ref,
               f3w_ref, f3b_ref, out_ref, acc_s, acc_c):
    pid = pl.program_id(0)
    nsteps = pl.num_programs(0)

    @pl.when(pid == 0)
    def _():
        acc_s[...] = jnp.zeros_like(acc_s)
        acc_c[...] = jnp.zeros_like(acc_c)

    seg = seg_ref[0, 0, :]
    blk = seg.shape[0]
    seg2 = jnp.broadcast_to(seg[None, :], (G, blk))
    gids = lax.broadcasted_iota(jnp.int32, (G, blk), 0)
    oh = (seg2 == gids).astype(jnp.float32)
    xb = x_ref[...]
    acc_s[...] += jnp.dot(oh, xb, preferred_element_type=jnp.float32,
                          precision=lax.Precision.HIGHEST)
    acc_c[...] += jnp.dot(oh, jnp.ones_like(xb),
                          preferred_element_type=jnp.float32)

    @pl.when(pid == nsteps - 1)
    def _():
        mean = acc_s[...] / jnp.maximum(acc_c[...], 1.0)
        z1 = _elu(jnp.dot(mean, f1w_ref[...],
                          preferred_element_type=jnp.float32) + f1b_ref[...])
        z2 = _elu(jnp.dot(z1, f2w_ref[...],
                          preferred_element_type=jnp.float32) + f2b_ref[...])
        out_ref[...] = (jnp.dot(z2, f3w_ref[...],
                                preferred_element_type=jnp.float32)
                        + f3b_ref[...])


@jax.jit
def _tc_head(seg3, x2, f1w, f1b, f2w, f2b, f3w8, f3b8):
    blk = 1024
    nblk = seg3.shape[0]
    return pl.pallas_call(
        _head_body,
        grid=(nblk,),
        in_specs=[
            pl.BlockSpec((1, 1, blk), lambda b: (b, 0, 0)),
            pl.BlockSpec((blk, 64), lambda b: (b, 0)),
            pl.BlockSpec((64, 32), lambda b: (0, 0)),
            pl.BlockSpec((1, 32), lambda b: (0, 0)),
            pl.BlockSpec((32, 16), lambda b: (0, 0)),
            pl.BlockSpec((1, 16), lambda b: (0, 0)),
            pl.BlockSpec((16, 8), lambda b: (0, 0)),
            pl.BlockSpec((1, 8), lambda b: (0, 0)),
        ],
        out_specs=pl.BlockSpec((G, 8), lambda b: (0, 0)),
        out_shape=jax.ShapeDtypeStruct((G, 8), jnp.float32),
        scratch_shapes=[
            pltpu.VMEM((G, 64), jnp.float32),
            pltpu.VMEM((G, 64), jnp.float32),
        ],
    )(seg3, x2, f1w, f1b, f2w, f2b, f3w8, f3b8)


def _nnconv_layer(x, src, dst, ea8, w1, b1, w2, b2, root, bias, tpad, blk):
    i, o = root.shape
    w18 = jnp.zeros((8, 128), jnp.float32).at[:5].set(w1)
    b1r = b1.reshape(1, 128)
    w2p = jnp.zeros((128, i, 128), jnp.float32)
    w2p = w2p.at[:, :, :o].set(w2.reshape(128, i, o)).reshape(128, i * 128)
    b2p = jnp.zeros((i, 128), jnp.float32)
    b2p = b2p.at[:, :o].set(b2.reshape(i, o)).reshape(1, i * 128)
    ep = src.shape[0]
    half = ep // 2
    aggs = []
    for lo in (0, half):
        srch = lax.slice_in_dim(src, lo, lo + half)
        dsth = lax.slice_in_dim(dst, lo, lo + half)
        eah = lax.slice_in_dim(ea8, lo, lo + half)
        xjh = _sc_gather(x, srch, half // 32)
        msgh = _tc_msg(xjh, eah, w18, b1r, w2p, b2p, blk, o)
        aggs.append(_sc_scatter_add(msgh, dsth, tpad, False))
    return _tc_combine2(aggs[0], aggs[1], x, root, bias.reshape(1, o), 512)


def kernel(x, edge_index, edge_attr, node_to_subgraph, original_edge_index,
           original_edge_attr, subgraph_to_graph,
           sub0_nw1, sub0_nb1, sub0_nw2, sub0_nb2, sub0_root, sub0_bias,
           sub1_nw1, sub1_nb1, sub1_nw2, sub1_nb2, sub1_root, sub1_bias,
           sub2_nw1, sub2_nb1, sub2_nw2, sub2_nb2, sub2_root, sub2_bias,
           gl0_nw1, gl0_nb1, gl0_nw2, gl0_nb2, gl0_root, gl0_bias,
           gl1_nw1, gl1_nb1, gl1_nw2, gl1_nb2, gl1_root, gl1_bias,
           fc1_w, fc1_b, fc2_w, fc2_b, fc3_w, fc3_b):
    f32 = jnp.float32
    x = x.astype(f32)

    src = jnp.pad(edge_index[0].astype(jnp.int32), (0, EP1 - E))
    dst = jnp.pad(edge_index[1].astype(jnp.int32), (0, EP1 - E),
                  constant_values=NPAD)
    ea8 = jnp.zeros((EP1, 8), f32).at[:E, :5].set(edge_attr)
    osrc = jnp.pad(original_edge_index[0].astype(jnp.int32), (0, EP2 - EO))
    odst = jnp.pad(original_edge_index[1].astype(jnp.int32), (0, EP2 - EO),
                   constant_values=SPAD)
    oea8 = jnp.zeros((EP2, 8), f32).at[:EO, :5].set(original_edge_attr)
    n2s = jnp.pad(node_to_subgraph.astype(jnp.int32), (0, NPAD - N),
                  constant_values=SPAD)
    s2g = jnp.pad(subgraph_to_graph.astype(jnp.int32), (0, SPAD - S),
                  constant_values=G)

    xp = jnp.zeros((NPAD, F_IN), f32).at[:N].set(x)

    subs = [
        (sub0_nw1, sub0_nb1, sub0_nw2, sub0_nb2, sub0_root, sub0_bias),
        (sub1_nw1, sub1_nb1, sub1_nw2, sub1_nb2, sub1_root, sub1_bias),
        (sub2_nw1, sub2_nb1, sub2_nw2, sub2_nb2, sub2_root, sub2_bias),
    ]
    h = xp
    for (w1, b1, w2, b2, root, bias) in subs:
        h = _nnconv_layer(h, src, dst, ea8, w1, b1, w2, b2, root, bias,
                          NPAD, 256)

    sums, counts = _sc_scatter_add(h, n2s, SPAD, True)
    p = _tc_divide(sums, counts)

    for (w1, b1, w2, b2, root, bias) in [
            (gl0_nw1, gl0_nb1, gl0_nw2, gl0_nb2, gl0_root, gl0_bias),
            (gl1_nw1, gl1_nb1, gl1_nw2, gl1_nb2, gl1_root, gl1_bias)]:
        p = _nnconv_layer(p, osrc, odst, oea8, w1, b1, w2, b2, root, bias,
                          SPAD, 256)

    seg3 = s2g.reshape(SPAD // 1024, 1, 1024)
    f3w8 = jnp.zeros((16, 8), f32).at[:, 0:1].set(fc3_w)
    f3b8 = jnp.zeros((1, 8), f32).at[0, 0].set(fc3_b[0])
    out8 = _tc_head(seg3, p, fc1_w, fc1_b.reshape(1, 32),
                    fc2_w, fc2_b.reshape(1, 16), f3w8, f3b8)
    return out8[:, 0]
